# baseline (speedup 1.0000x reference)
import functools

import jax
import jax.numpy as jnp
from jax import lax
from jax.experimental import pallas as pl
from jax.experimental.pallas import tpu as pltpu
from jax.experimental.pallas import tpu_sc as plsc

_B, _S, _H, _V, _P = 64, 512, 1024, 50265, 50
_PAD = 1
_EPS = 1e-5

_NC, _NS = 2, 16
_NW = _NC * _NS
_SWID = _S // _NW
_NRING = 4
_UNROLL = 16

_sc_mesh = plsc.VectorSubcoreMesh(core_axis_name="c", subcore_axis_name="s")


def _xsum(x):
    idx = lax.broadcasted_iota(jnp.int32, (16,), 0)
    dnums = lax.GatherDimensionNumbers(
        offset_dims=(), collapsed_slice_dims=(0,), start_index_map=(0,))
    for sft in (8, 4, 2, 1):
        perm = lax.bitwise_xor(idx, jnp.int32(sft)).reshape(16, 1)
        x = x + lax.gather(x, perm, dnums, (1,),
                           mode=lax.GatherScatterMode.PROMISE_IN_BOUNDS)
    return x


def _vrsqrt(v):
    i = lax.bitcast_convert_type(v, jnp.int32)
    i = jnp.int32(0x5F3759DF) - lax.shift_right_logical(i, 1)
    y = lax.bitcast_convert_type(i, jnp.float32)
    for _ in range(3):
        y = y * (1.5 - 0.5 * v * y * y)
    return y


@functools.partial(
    pl.kernel,
    mesh=_sc_mesh,
    out_type=jax.ShapeDtypeStruct((_B * _S, _H), jnp.float32),
    scratch_types=[
        pltpu.VMEM((_B * _SWID,), jnp.int32),
        pltpu.VMEM((_SWID, _H), jnp.float32),
        pltpu.VMEM((_SWID, _H), jnp.float32),
        pltpu.VMEM((_SWID, _H), jnp.float32),
        pltpu.VMEM((_SWID, _H), jnp.float32),
        pltpu.VMEM((_SWID, _H), jnp.float32),
        pltpu.VMEM((_SWID, _H), jnp.float32),
        pltpu.VMEM((_SWID, _H), jnp.float32),
        pltpu.VMEM((_H,), jnp.float32),
        pltpu.VMEM((_H,), jnp.float32),
        pltpu.SemaphoreType.DMA,
        pltpu.SemaphoreType.DMA,
        pltpu.SemaphoreType.DMA,
        pltpu.SemaphoreType.DMA,
        pltpu.SemaphoreType.DMA,
        pltpu.SemaphoreType.DMA,
    ],
)
def _sc_fused(ids_hbm, table_hbm, padd_hbm, g_hbm, bt_hbm, out_hbm,
              idx_v, x0, x1, x2, x3, pbuf, o0, o1, gbuf, bbuf,
              gs0, gs1, gs2, gs3, ws0, ws1):
    wid = lax.axis_index("s") * _NC + lax.axis_index("c")
    s0 = wid * _SWID

    xb = (x0, x1, x2, x3)
    ob = (o0, o1)
    gs = (gs0, gs1, gs2, gs3)
    ws = (ws0, ws1)

    def gdesc(b, m):
        return pltpu.make_async_copy(
            table_hbm.at[idx_v.at[pl.ds(b * _SWID, _SWID)]], xb[m], gs[m])

    def wdesc(b, pm):
        return pltpu.make_async_copy(
            ob[pm], out_hbm.at[pl.ds(b * _S + s0, _SWID)], ws[pm])

    pltpu.sync_copy(ids_hbm.at[pl.ds(wid * (_B * _SWID), _B * _SWID)], idx_v)
    pltpu.sync_copy(padd_hbm.at[pl.ds(s0, _SWID)], pbuf)
    pltpu.sync_copy(g_hbm, gbuf)
    pltpu.sync_copy(bt_hbm, bbuf)

    for m in range(_NRING):
        gdesc(m, m).start()

    zeros = jnp.zeros((16,), jnp.float32)
    inv_h = jnp.float32(1.0 / _H)
    mf = [jnp.where(s0 + sl >= _P, jnp.float32(1.0), jnp.float32(0.0))
          for sl in range(_SWID)]
    _HALF = _SWID // 2

    def outer(k, carry):
        for m in range(_NRING):
            b = _NRING * k + m
            pm = m % 2
            gdesc(b, m).wait()

            means = []
            rstds = []
            for half in range(2):
                sls = list(range(half * _HALF, (half + 1) * _HALF))

                def pass1(t, c, m=m, sls=sls):
                    cl = list(c)
                    for i, sl in enumerate(sls):
                        for u in range(_UNROLL):
                            off = (_UNROLL * t + u) * 16
                            x = xb[m][sl, pl.ds(off, 16)]
                            p = pbuf[sl, pl.ds(off, 16)]
                            y = x * mf[sl] + p
                            xb[m][sl, pl.ds(off, 16)] = y
                            cl[2 * i] = cl[2 * i] + y
                            cl[2 * i + 1] = cl[2 * i + 1] + y * y
                    return tuple(cl)

                res = lax.fori_loop(0, _H // (16 * _UNROLL), pass1,
                                    (zeros,) * (2 * _HALF))
                for i in range(_HALF):
                    mean = _xsum(res[2 * i]) * inv_h
                    var = _xsum(res[2 * i + 1]) * inv_h - mean * mean
                    means.append(mean)
                    rstds.append(_vrsqrt(var + _EPS))

            if m >= 2:
                wdesc(b, pm).wait()
            else:
                @pl.when(k > 0)
                def _():
                    wdesc(b, pm).wait()

            def pass2(j, c, m=m, pm=pm):
                for u in range(2):
                    jj = 2 * j + u
                    gj = gbuf[pl.ds(16 * jj, 16)]
                    bj = bbuf[pl.ds(16 * jj, 16)]
                    for sl in range(_SWID):
                        y = xb[m][sl, pl.ds(16 * jj, 16)]
                        ob[pm][sl, pl.ds(16 * jj, 16)] = (
                            (y - means[sl]) * rstds[sl] * gj + bj)
                return c

            lax.fori_loop(0, _H // 32, pass2, 0)
            wdesc(b, pm).start()

            @pl.when(k < _B // _NRING - 1)
            def _(b=b, m=m):
                gdesc(b + _NRING, m).start()

        return carry

    lax.fori_loop(0, _B // _NRING, outer, 0)
    wdesc(0, 0).wait()
    wdesc(0, 1).wait()


def _mlp_padd_body(p_ref, w1_ref, b1_ref, w2_ref, b2_ref, pos_ref, type_ref,
                   o_ref):
    h = jnp.dot(p_ref[...], w1_ref[...], preferred_element_type=jnp.float32)
    h = jnp.maximum(h + b1_ref[...], 0.0)
    mlp = jnp.dot(h, w2_ref[...], preferred_element_type=jnp.float32) + b2_ref[...]
    r = lax.broadcasted_iota(jnp.int32, (_S, 1), 0)
    o_ref[...] = jnp.where(r < _P, mlp, 0.0) + pos_ref[...] + type_ref[...]


def kernel(input_ids, prompt_pos, word_table, prompt_table, W1, b1, W2, b2,
           pos_table, type_table, ln_gamma, ln_beta):
    ids_w = (input_ids.astype(jnp.int32)
             .reshape(_B, _NW, _SWID)
             .transpose(1, 0, 2)
             .reshape(_B * _S))

    p_pad = jnp.zeros((_S, _H), jnp.float32).at[:_P].set(prompt_table)
    pos_slice = lax.slice(pos_table, (_PAD + 1, 0), (_PAD + 1 + _S, _H))
    padd = pl.pallas_call(
        _mlp_padd_body,
        out_shape=jax.ShapeDtypeStruct((_S, _H), jnp.float32),
    )(p_pad, W1, b1.reshape(1, _H), W2, b2.reshape(1, _H), pos_slice,
      type_table)

    out = _sc_fused(ids_w, word_table, padd, ln_gamma, ln_beta)
    return out.reshape(_B, _S, _H)


# speedup vs baseline: 1.8686x; 1.8686x over previous
import functools

import jax
import jax.numpy as jnp
from jax import lax
from jax.experimental import pallas as pl
from jax.experimental.pallas import tpu as pltpu
from jax.experimental.pallas import tpu_sc as plsc

_B, _S, _H, _V, _P = 64, 512, 1024, 50265, 50
_PAD = 1
_EPS = 1e-5

_NC, _NS = 2, 16
_NW = _NC * _NS
_ROWS = _B * _S
_NSPLIT = 4
_ROWS_C = _ROWS // _NSPLIT
_RPW = _ROWS_C // _NW
_CH = 32
_NCHUNK = _RPW // _CH

_sc_mesh = plsc.VectorSubcoreMesh(core_axis_name="c", subcore_axis_name="s")


@functools.partial(
    pl.kernel,
    mesh=_sc_mesh,
    out_type=jax.ShapeDtypeStruct((_ROWS_C, _H), jnp.float32),
    scratch_types=[
        pltpu.VMEM((_RPW,), jnp.int32),
        pltpu.VMEM((_CH, _H), jnp.float32),
        pltpu.VMEM((_CH, _H), jnp.float32),
        pltpu.SemaphoreType.DMA,
        pltpu.SemaphoreType.DMA,
    ],
)
def _sc_gather(ids_hbm, table_hbm, out_hbm, idx_v, rows0_v, rows1_v, sem0, sem1):
    wid = lax.axis_index("s") * _NC + lax.axis_index("c")
    base = wid * _RPW

    def gather(c, buf, sem):
        return pltpu.make_async_copy(
            table_hbm.at[idx_v.at[pl.ds(c * _CH, _CH)]], buf, sem)

    def writeback(c, buf):
        pltpu.sync_copy(buf, out_hbm.at[pl.ds(base + c * _CH, _CH)])

    pltpu.sync_copy(ids_hbm.at[pl.ds(base, _RPW)], idx_v)
    gather(0, rows0_v, sem0).start()

    def body(k, carry):
        c0 = 2 * k
        gather(c0 + 1, rows1_v, sem1).start()
        gather(c0, rows0_v, sem0).wait()
        writeback(c0, rows0_v)

        @pl.when(k < _NCHUNK // 2 - 1)
        def _():
            gather(c0 + 2, rows0_v, sem0).start()

        gather(c0 + 1, rows1_v, sem1).wait()
        writeback(c0 + 1, rows1_v)
        return carry

    lax.fori_loop(0, _NCHUNK // 2, body, 0)


def _mlp_body(p_ref, w1_ref, b1_ref, w2_ref, b2_ref, o_ref):
    h = jnp.dot(p_ref[...], w1_ref[...], preferred_element_type=jnp.float32)
    h = jnp.maximum(h + b1_ref[...], 0.0)
    o_ref[...] = jnp.dot(h, w2_ref[...], preferred_element_type=jnp.float32) + b2_ref[...]


_BLK = 512


def _ln_combine(raw_ref, pos_ref, pe_ref, type_ref, g_ref, b_ref, o_ref):
    r = lax.broadcasted_iota(jnp.int32, (_BLK, 1), 0)
    mask = r < _P
    x = jnp.where(mask, pe_ref[...], raw_ref[...])
    x = x + pos_ref[...] + type_ref[...]
    mean = jnp.mean(x, axis=1, keepdims=True)
    cent = x - mean
    var = jnp.mean(cent * cent, axis=1, keepdims=True)
    o_ref[...] = cent * lax.rsqrt(var + _EPS) * g_ref[...] + b_ref[...]


def _combine_first(raw_ref, pos_ref, pe_ref, type_ref, g_ref, b_ref, o_ref):
    _ln_combine(raw_ref, pos_ref, pe_ref, type_ref, g_ref, b_ref, o_ref)


def _combine_chained(raw_ref, pos_ref, pe_ref, type_ref, g_ref, b_ref,
                     prev_ref, o_ref):
    _ln_combine(raw_ref, pos_ref, pe_ref, type_ref, g_ref, b_ref, o_ref)


_BPC = _ROWS_C // _BLK


def _combine_call(c, raw_c, pos_slice, pe, type_table, g2d, b2d, prev):
    base_specs = [
        pl.BlockSpec((_BLK, _H), lambda i: (i, 0)),
        pl.BlockSpec((_BLK, _H), lambda i: (0, 0)),
        pl.BlockSpec((_BLK, _H), lambda i: (0, 0)),
        pl.BlockSpec((1, _H), lambda i: (0, 0)),
        pl.BlockSpec((1, _H), lambda i: (0, 0)),
        pl.BlockSpec((1, _H), lambda i: (0, 0)),
    ]
    out_spec = pl.BlockSpec((_BLK, _H), lambda i, c=c: (c * _BPC + i, 0))
    out_shape = jax.ShapeDtypeStruct((_ROWS, _H), jnp.float32)
    args = (raw_c, pos_slice, pe, type_table, g2d, b2d)
    if prev is None:
        return pl.pallas_call(
            _combine_first, grid=(_BPC,), in_specs=base_specs,
            out_specs=out_spec, out_shape=out_shape)(*args)
    return pl.pallas_call(
        _combine_chained, grid=(_BPC,),
        in_specs=base_specs + [pl.BlockSpec(memory_space=pl.ANY)],
        out_specs=out_spec, out_shape=out_shape,
        input_output_aliases={6: 0})(*args, prev)


def kernel(input_ids, prompt_pos, word_table, prompt_table, W1, b1, W2, b2,
           pos_table, type_table, ln_gamma, ln_beta):
    ids_flat = input_ids.reshape(_ROWS).astype(jnp.int32)

    p_pad = jnp.zeros((_BLK, _H), jnp.float32).at[:_P].set(prompt_table)
    pe = pl.pallas_call(
        _mlp_body,
        out_shape=jax.ShapeDtypeStruct((_BLK, _H), jnp.float32),
    )(p_pad, W1, b1.reshape(1, _H), W2, b2.reshape(1, _H))

    pos_slice = lax.slice(pos_table, (_PAD + 1, 0), (_PAD + 1 + _S, _H))
    g2d = ln_gamma.reshape(1, _H)
    b2d = ln_beta.reshape(1, _H)

    out = None
    for c in range(_NSPLIT):
        ids_c = lax.slice(ids_flat, (c * _ROWS_C,), ((c + 1) * _ROWS_C,))
        raw_c = _sc_gather(ids_c, word_table)
        out = _combine_call(c, raw_c, pos_slice, pe, type_table, g2d, b2d, out)

    return out.reshape(_B, _S, _H)


# speedup vs baseline: 1.8715x; 1.0015x over previous
import functools

import jax
import jax.numpy as jnp
from jax import lax
from jax.experimental import pallas as pl
from jax.experimental.pallas import tpu as pltpu
from jax.experimental.pallas import tpu_sc as plsc

_B, _S, _H, _V, _P = 64, 512, 1024, 50265, 50
_PAD = 1
_EPS = 1e-5

_NC, _NS = 2, 16
_NW = _NC * _NS
_ROWS = _B * _S
_NSPLIT = 2
_ROWS_C = _ROWS // _NSPLIT
_RPW = _ROWS_C // _NW
_CH = 32
_NCHUNK = _RPW // _CH

_sc_mesh = plsc.VectorSubcoreMesh(core_axis_name="c", subcore_axis_name="s")


@functools.partial(
    pl.kernel,
    mesh=_sc_mesh,
    out_type=jax.ShapeDtypeStruct((_ROWS_C, _H), jnp.float32),
    scratch_types=[
        pltpu.VMEM((_RPW,), jnp.int32),
        pltpu.VMEM((_CH, _H), jnp.float32),
        pltpu.VMEM((_CH, _H), jnp.float32),
        pltpu.SemaphoreType.DMA,
        pltpu.SemaphoreType.DMA,
    ],
)
def _sc_gather(ids_hbm, table_hbm, out_hbm, idx_v, rows0_v, rows1_v, sem0, sem1):
    wid = lax.axis_index("s") * _NC + lax.axis_index("c")
    base = wid * _RPW

    def gather(c, buf, sem):
        return pltpu.make_async_copy(
            table_hbm.at[idx_v.at[pl.ds(c * _CH, _CH)]], buf, sem)

    def writeback(c, buf):
        pltpu.sync_copy(buf, out_hbm.at[pl.ds(base + c * _CH, _CH)])

    pltpu.sync_copy(ids_hbm.at[pl.ds(base, _RPW)], idx_v)
    gather(0, rows0_v, sem0).start()

    def body(k, carry):
        c0 = 2 * k
        gather(c0 + 1, rows1_v, sem1).start()
        gather(c0, rows0_v, sem0).wait()
        writeback(c0, rows0_v)

        @pl.when(k < _NCHUNK // 2 - 1)
        def _():
            gather(c0 + 2, rows0_v, sem0).start()

        gather(c0 + 1, rows1_v, sem1).wait()
        writeback(c0 + 1, rows1_v)
        return carry

    lax.fori_loop(0, _NCHUNK // 2, body, 0)


def _mlp_body(p_ref, w1_ref, b1_ref, w2_ref, b2_ref, o_ref):
    h = jnp.dot(p_ref[...], w1_ref[...], preferred_element_type=jnp.float32)
    h = jnp.maximum(h + b1_ref[...], 0.0)
    o_ref[...] = jnp.dot(h, w2_ref[...], preferred_element_type=jnp.float32) + b2_ref[...]


_BLK = 512


def _ln_combine(raw_ref, pos_ref, pe_ref, type_ref, g_ref, b_ref, o_ref):
    r = lax.broadcasted_iota(jnp.int32, (_BLK, 1), 0)
    mask = r < _P
    x = jnp.where(mask, pe_ref[...], raw_ref[...])
    x = x + pos_ref[...] + type_ref[...]
    mean = jnp.mean(x, axis=1, keepdims=True)
    cent = x - mean
    var = jnp.mean(cent * cent, axis=1, keepdims=True)
    o_ref[...] = cent * lax.rsqrt(var + _EPS) * g_ref[...] + b_ref[...]


def _combine_first(raw_ref, pos_ref, pe_ref, type_ref, g_ref, b_ref, o_ref):
    _ln_combine(raw_ref, pos_ref, pe_ref, type_ref, g_ref, b_ref, o_ref)


def _combine_chained(raw_ref, pos_ref, pe_ref, type_ref, g_ref, b_ref,
                     prev_ref, o_ref):
    _ln_combine(raw_ref, pos_ref, pe_ref, type_ref, g_ref, b_ref, o_ref)


_BPC = _ROWS_C // _BLK


def _combine_call(c, raw_c, pos_slice, pe, type_table, g2d, b2d, prev):
    base_specs = [
        pl.BlockSpec((_BLK, _H), lambda i: (i, 0)),
        pl.BlockSpec((_BLK, _H), lambda i: (0, 0)),
        pl.BlockSpec((_BLK, _H), lambda i: (0, 0)),
        pl.BlockSpec((1, _H), lambda i: (0, 0)),
        pl.BlockSpec((1, _H), lambda i: (0, 0)),
        pl.BlockSpec((1, _H), lambda i: (0, 0)),
    ]
    out_spec = pl.BlockSpec((_BLK, _H), lambda i, c=c: (c * _BPC + i, 0))
    out_shape = jax.ShapeDtypeStruct((_ROWS, _H), jnp.float32)
    args = (raw_c, pos_slice, pe, type_table, g2d, b2d)
    if prev is None:
        return pl.pallas_call(
            _combine_first, grid=(_BPC,), in_specs=base_specs,
            out_specs=out_spec, out_shape=out_shape)(*args)
    return pl.pallas_call(
        _combine_chained, grid=(_BPC,),
        in_specs=base_specs + [pl.BlockSpec(memory_space=pl.ANY)],
        out_specs=out_spec, out_shape=out_shape,
        input_output_aliases={6: 0})(*args, prev)


def kernel(input_ids, prompt_pos, word_table, prompt_table, W1, b1, W2, b2,
           pos_table, type_table, ln_gamma, ln_beta):
    ids_flat = input_ids.reshape(_ROWS).astype(jnp.int32)

    p_pad = jnp.zeros((_BLK, _H), jnp.float32).at[:_P].set(prompt_table)
    pe = pl.pallas_call(
        _mlp_body,
        out_shape=jax.ShapeDtypeStruct((_BLK, _H), jnp.float32),
    )(p_pad, W1, b1.reshape(1, _H), W2, b2.reshape(1, _H))

    pos_slice = lax.slice(pos_table, (_PAD + 1, 0), (_PAD + 1 + _S, _H))
    g2d = ln_gamma.reshape(1, _H)
    b2d = ln_beta.reshape(1, _H)

    out = None
    for c in range(_NSPLIT):
        ids_c = lax.slice(ids_flat, (c * _ROWS_C,), ((c + 1) * _ROWS_C,))
        raw_c = _sc_gather(ids_c, word_table)
        out = _combine_call(c, raw_c, pos_slice, pe, type_table, g2d, b2d, out)

    return out.reshape(_B, _S, _H)
